# baseline (speedup 1.0000x reference)
import functools

import jax
import jax.numpy as jnp
from jax import lax
from jax.experimental import pallas as pl
from jax.experimental.pallas import tpu as pltpu
from jax.experimental.pallas import tpu_sc as plsc

N_NODES = 10000
DEG = 16
D = 256
B = 10000
BP = 10240
NC = 2
NS = 16
NW = NC * NS
BPW = BP // NW
CH = 8
NCHUNK = BPW // CH


def _sc_gather_kernel(nodes_h, adj_h, emb_h, self_h, nsum_h,
                      idxc, adjc0, adjc1, nidx_all,
                      sbuf0, sbuf1, nbuf0, nbuf1, mbuf0, mbuf1,
                      sga0, sga1, sgn0, sgn1, sgs0, sgs1,
                      swm0, swm1, sws0, sws1):
    cid = lax.axis_index("c")
    sid = lax.axis_index("s")
    wid = sid * NC + cid
    base = wid * BPW

    pltpu.sync_copy(nodes_h.at[pl.ds(wid * NCHUNK, NCHUNK)], idxc)

    def adj_dma(c, buf, sem):
        return pltpu.make_async_copy(adj_h.at[idxc.at[c]], buf, sem)

    def stage_nidx(c, buf):
        for r in range(CH):
            nidx_all[c, pl.ds(r * DEG, DEG)] = buf[r, pl.ds(0, DEG)]

    adj_dma(0, adjc0, sga0).start()

    def phase1(i, carry):
        c = i * 2
        adj_dma(c + 1, adjc1, sga1).start()
        adj_dma(c, adjc0, sga0).wait()
        stage_nidx(c, adjc0)

        @pl.when(c + 2 < NCHUNK)
        def _():
            adj_dma(c + 2, adjc0, sga0).start()

        adj_dma(c + 1, adjc1, sga1).wait()
        stage_nidx(c + 1, adjc1)
        return carry

    lax.fori_loop(0, NCHUNK // 2, phase1, 0)

    def n_dma(c, nb, sem):
        return pltpu.make_async_copy(emb_h.at[nidx_all.at[c]], nb, sem)

    def s_dma(c, sb, sem):
        return pltpu.make_async_copy(emb_h.at[idxc.at[c]], sb, sem)

    def wm_dma(c, mb, sem):
        return pltpu.make_async_copy(mb, nsum_h.at[pl.ds(base + c * CH, CH)], sem)

    def ws_dma(c, sb, sem):
        return pltpu.make_async_copy(sb, self_h.at[pl.ds(base + c * CH, CH)], sem)

    def reduce_chunk(nb, mb):
        himask = jnp.full((16,), -65536, jnp.int32)

        bc = lambda v: lax.bitcast_convert_type(v, jnp.float32)

        def row(r, carry):
            for k in range(8):
                cs = pl.ds(k * 16, 16)
                w = nb[r * DEG, cs]
                alo = bc(w << 16)
                ahi = bc(w & himask)
                for j in range(1, DEG):
                    w = nb[r * DEG + j, cs]
                    alo = alo + bc(w << 16)
                    ahi = ahi + bc(w & himask)
                mb[r, cs] = alo
                mb[r, pl.ds(128 + k * 16, 16)] = ahi
            return carry
        lax.fori_loop(0, CH, row, 0)

    n_dma(0, nbuf0, sgn0).start()
    s_dma(0, sbuf0, sgs0).start()

    def half(c, nb, sb, mb, sgn, sgs, swm, sws, nb_n, sb_n, sgn_n, sgs_n,
             sws_n):
        @pl.when(c + 1 < NCHUNK)
        def _():
            @pl.when(c >= 1)
            def _():
                ws_dma(c - 1, sb_n, sws_n).wait()
            n_dma(c + 1, nb_n, sgn_n).start()
            s_dma(c + 1, sb_n, sgs_n).start()

        n_dma(c, nb, sgn).wait()
        s_dma(c, sb, sgs).wait()
        ws_dma(c, sb, sws).start()

        @pl.when(c >= 2)
        def _():
            wm_dma(c - 2, mb, swm).wait()
        reduce_chunk(nb, mb)
        wm_dma(c, mb, swm).start()

    def phase2(i, carry):
        c = i * 2
        half(c, nbuf0, sbuf0, mbuf0, sgn0, sgs0, swm0, sws0,
             nbuf1, sbuf1, sgn1, sgs1, sws1)
        half(c + 1, nbuf1, sbuf1, mbuf1, sgn1, sgs1, swm1, sws1,
             nbuf0, sbuf0, sgn0, sgs0, sws0)
        return carry

    lax.fori_loop(0, NCHUNK // 2, phase2, 0)

    wm_dma(NCHUNK - 2, mbuf0, swm0).wait()
    ws_dma(NCHUNK - 2, sbuf0, sws0).wait()
    wm_dma(NCHUNK - 1, mbuf1, swm1).wait()
    ws_dma(NCHUNK - 1, sbuf1, sws1).wait()


def _sc_gather(nodes_p, adj_p, emb_b3):
    mesh = plsc.VectorSubcoreMesh(core_axis_name="c", subcore_axis_name="s")
    kern = functools.partial(
        pl.kernel,
        mesh=mesh,
        out_type=(
            jax.ShapeDtypeStruct((BP, 128), jnp.int32),
            jax.ShapeDtypeStruct((BP, D), jnp.float32),
        ),
        scratch_types=[
            pltpu.VMEM((NCHUNK, CH), jnp.int32),
            pltpu.VMEM((CH, 128), jnp.int32),
            pltpu.VMEM((CH, 128), jnp.int32),
            pltpu.VMEM((NCHUNK, CH * DEG), jnp.int32),
            pltpu.VMEM((CH, 128), jnp.int32),
            pltpu.VMEM((CH, 128), jnp.int32),
            pltpu.VMEM((CH * DEG, 128), jnp.int32),
            pltpu.VMEM((CH * DEG, 128), jnp.int32),
            pltpu.VMEM((CH, D), jnp.float32),
            pltpu.VMEM((CH, D), jnp.float32),
        ] + [pltpu.SemaphoreType.DMA] * 10,
    )(_sc_gather_kernel)
    return kern(nodes_p, adj_p, emb_b3)


def _mm_kernel(x1_ref, x2_ref, w1_ref, w2_ref, b_ref, o_ref):
    acc = jnp.dot(x1_ref[...], w1_ref[...], preferred_element_type=jnp.float32)
    acc = acc + jnp.dot(x2_ref[...], w2_ref[...], preferred_element_type=jnp.float32)
    o_ref[...] = jnp.maximum(acc + b_ref[...], 0.0)


def _tc_matmul(self_f, nsum, w1, w2, b2):
    bm = 1000
    grid = (B // bm,)
    return pl.pallas_call(
        _mm_kernel,
        grid=grid,
        in_specs=[
            pl.BlockSpec((bm, D), lambda i: (i, 0)),
            pl.BlockSpec((bm, D), lambda i: (i, 0)),
            pl.BlockSpec((D, D), lambda i: (0, 0)),
            pl.BlockSpec((D, D), lambda i: (0, 0)),
            pl.BlockSpec((1, D), lambda i: (0, 0)),
        ],
        out_specs=pl.BlockSpec((bm, D), lambda i: (i, 0)),
        out_shape=jax.ShapeDtypeStruct((B, D), jnp.float32),
    )(self_f, nsum, w1, w2, b2)


def kernel(nodes, adj, emb, W, b):
    nodes_p = jnp.pad(nodes.astype(jnp.int32), (0, BP - B)).reshape(NW * NCHUNK, CH)
    adj_p = jnp.pad(adj.astype(jnp.int32), ((0, 0), (0, 128 - DEG)))
    emb_i = lax.bitcast_convert_type(
        emb.astype(jnp.bfloat16).reshape(N_NODES, D // 2, 2), jnp.int32)
    self_i, nsum = _sc_gather(nodes_p, adj_p, emb_i)
    self_b = lax.bitcast_convert_type(self_i, jnp.bfloat16).reshape(BP, D)
    w1 = W[:D, :].astype(jnp.bfloat16)
    w2 = W[D:, :] * (1.0 / DEG)
    w2p = jnp.concatenate([w2[0::2, :], w2[1::2, :]], axis=0)
    return _tc_matmul(self_b, nsum, w1, w2p, b.reshape(1, D))


# speedup vs baseline: 2.2061x; 2.2061x over previous
import functools

import jax
import jax.numpy as jnp
from jax import lax
from jax.experimental import pallas as pl
from jax.experimental.pallas import tpu as pltpu
from jax.experimental.pallas import tpu_sc as plsc

N_NODES = 10000
DEG = 16
D = 256
B = 10000
BP = 10240
NC = 2
NS = 16
NW = NC * NS
BPW = BP // NW
CH = 8
NCHUNK = BPW // CH


def _sc_gather_kernel(nodes_h, adj_h, emb_h, self_h, nsum_h,
                      idxc, adjc0, adjc1, nidx_all,
                      sbuf0, sbuf1, nbuf0, nbuf1, mbuf0, mbuf1,
                      sga0, sga1, sgn0, sgn1, sgs0, sgs1,
                      swm0, swm1, sws0, sws1):
    cid = lax.axis_index("c")
    sid = lax.axis_index("s")
    wid = sid * NC + cid
    base = wid * BPW

    pltpu.sync_copy(nodes_h.at[pl.ds(wid * NCHUNK, NCHUNK)], idxc)

    def adj_dma(c, buf, sem):
        return pltpu.make_async_copy(adj_h.at[idxc.at[c]], buf, sem)

    def stage_nidx(c, buf):
        for r in range(CH):
            nidx_all[c, pl.ds(r * DEG, DEG)] = buf[r, pl.ds(0, DEG)]

    adj_dma(0, adjc0, sga0).start()

    def phase1(i, carry):
        c = i * 2
        adj_dma(c + 1, adjc1, sga1).start()
        adj_dma(c, adjc0, sga0).wait()
        stage_nidx(c, adjc0)

        @pl.when(c + 2 < NCHUNK)
        def _():
            adj_dma(c + 2, adjc0, sga0).start()

        adj_dma(c + 1, adjc1, sga1).wait()
        stage_nidx(c + 1, adjc1)
        return carry

    lax.fori_loop(0, NCHUNK // 2, phase1, 0)

    def n_dma(c, nb, sem):
        return pltpu.make_async_copy(emb_h.at[nidx_all.at[c]], nb, sem)

    def s_dma(c, sb, sem):
        return pltpu.make_async_copy(emb_h.at[idxc.at[c]], sb, sem)

    def wm_dma(c, mb, sem):
        return pltpu.make_async_copy(mb, nsum_h.at[pl.ds(base + c * CH, CH)], sem)

    def ws_dma(c, sb, sem):
        return pltpu.make_async_copy(sb, self_h.at[pl.ds(base + c * CH, CH)], sem)

    def reduce_chunk(nb, mb):
        bc = lambda v: lax.bitcast_convert_type(v, jnp.float32)

        def row(r, carry):
            for k in range(8):
                cs = pl.ds(k * 16, 16)
                w = nb[r * DEG, cs]
                alo = bc(w << 16)
                ahi = bc(w)
                for j in range(1, DEG):
                    w = nb[r * DEG + j, cs]
                    alo = alo + bc(w << 16)
                    ahi = ahi + bc(w)
                mb[r, cs] = alo
                mb[r, pl.ds(128 + k * 16, 16)] = ahi
            return carry
        lax.fori_loop(0, CH, row, 0)

    n_dma(0, nbuf0, sgn0).start()
    s_dma(0, sbuf0, sgs0).start()

    def half(c, nb, sb, mb, sgn, sgs, swm, sws, nb_n, sb_n, sgn_n, sgs_n,
             sws_n):
        @pl.when(c + 1 < NCHUNK)
        def _():
            @pl.when(c >= 1)
            def _():
                ws_dma(c - 1, sb_n, sws_n).wait()
            n_dma(c + 1, nb_n, sgn_n).start()
            s_dma(c + 1, sb_n, sgs_n).start()

        n_dma(c, nb, sgn).wait()
        s_dma(c, sb, sgs).wait()
        ws_dma(c, sb, sws).start()

        @pl.when(c >= 2)
        def _():
            wm_dma(c - 2, mb, swm).wait()
        reduce_chunk(nb, mb)
        wm_dma(c, mb, swm).start()

    def phase2(i, carry):
        c = i * 2
        half(c, nbuf0, sbuf0, mbuf0, sgn0, sgs0, swm0, sws0,
             nbuf1, sbuf1, sgn1, sgs1, sws1)
        half(c + 1, nbuf1, sbuf1, mbuf1, sgn1, sgs1, swm1, sws1,
             nbuf0, sbuf0, sgn0, sgs0, sws0)
        return carry

    lax.fori_loop(0, NCHUNK // 2, phase2, 0)

    wm_dma(NCHUNK - 2, mbuf0, swm0).wait()
    ws_dma(NCHUNK - 2, sbuf0, sws0).wait()
    wm_dma(NCHUNK - 1, mbuf1, swm1).wait()
    ws_dma(NCHUNK - 1, sbuf1, sws1).wait()


def _sc_gather(nodes_p, adj_p, emb_b3):
    mesh = plsc.VectorSubcoreMesh(core_axis_name="c", subcore_axis_name="s")
    kern = functools.partial(
        pl.kernel,
        mesh=mesh,
        out_type=(
            jax.ShapeDtypeStruct((BP, 128), jnp.int32),
            jax.ShapeDtypeStruct((BP, D), jnp.float32),
        ),
        scratch_types=[
            pltpu.VMEM((NCHUNK, CH), jnp.int32),
            pltpu.VMEM((CH, 128), jnp.int32),
            pltpu.VMEM((CH, 128), jnp.int32),
            pltpu.VMEM((NCHUNK, CH * DEG), jnp.int32),
            pltpu.VMEM((CH, 128), jnp.int32),
            pltpu.VMEM((CH, 128), jnp.int32),
            pltpu.VMEM((CH * DEG, 128), jnp.int32),
            pltpu.VMEM((CH * DEG, 128), jnp.int32),
            pltpu.VMEM((CH, D), jnp.float32),
            pltpu.VMEM((CH, D), jnp.float32),
        ] + [pltpu.SemaphoreType.DMA] * 10,
    )(_sc_gather_kernel)
    return kern(nodes_p, adj_p, emb_b3)


def _mm_kernel(x1_ref, x2_ref, w1a_ref, w1b_ref, w2_ref, b_ref, o_ref):
    x = x1_ref[...]
    xlo = lax.bitcast_convert_type(x << 16, jnp.float32).astype(jnp.bfloat16)
    xhi = lax.bitcast_convert_type(x & jnp.int32(-65536), jnp.float32
                                   ).astype(jnp.bfloat16)
    acc = jnp.dot(xlo, w1a_ref[...], preferred_element_type=jnp.float32)
    acc += jnp.dot(xhi, w1b_ref[...], preferred_element_type=jnp.float32)
    acc += jnp.dot(x2_ref[...].astype(jnp.bfloat16), w2_ref[...],
                   preferred_element_type=jnp.float32)
    o_ref[...] = jnp.maximum(acc + b_ref[...], 0.0)


def _tc_matmul(self_i, nsum, w1a, w1b, w2, b2):
    bm = 1000
    grid = (B // bm,)
    return pl.pallas_call(
        _mm_kernel,
        grid=grid,
        in_specs=[
            pl.BlockSpec((bm, D // 2), lambda i: (i, 0)),
            pl.BlockSpec((bm, D), lambda i: (i, 0)),
            pl.BlockSpec((D // 2, D), lambda i: (0, 0)),
            pl.BlockSpec((D // 2, D), lambda i: (0, 0)),
            pl.BlockSpec((D, D), lambda i: (0, 0)),
            pl.BlockSpec((1, D), lambda i: (0, 0)),
        ],
        out_specs=pl.BlockSpec((bm, D), lambda i: (i, 0)),
        out_shape=jax.ShapeDtypeStruct((B, D), jnp.float32),
    )(self_i, nsum, w1a, w1b, w2, b2)


def kernel(nodes, adj, emb, W, b):
    nodes_p = jnp.pad(nodes.astype(jnp.int32), (0, BP - B)).reshape(NW * NCHUNK, CH)
    adj_p = jnp.pad(adj.astype(jnp.int32), ((0, 0), (0, 128 - DEG)))
    u = lax.bitcast_convert_type(emb.astype(jnp.bfloat16), jnp.uint16)
    emb_i = (u[:, :D // 2].astype(jnp.uint32)
             | (u[:, D // 2:].astype(jnp.uint32) << 16)).astype(jnp.int32)
    self_i, nsum = _sc_gather(nodes_p, adj_p, emb_i)
    w1 = W[:D, :].astype(jnp.bfloat16)
    w1a, w1b = w1[:D // 2, :], w1[D // 2:, :]
    w2 = (W[D:, :] * (1.0 / DEG)).astype(jnp.bfloat16)
    return _tc_matmul(self_i, nsum, w1a, w1b, w2, b.reshape(1, D))


# speedup vs baseline: 2.5911x; 1.1745x over previous
import functools

import jax
import jax.numpy as jnp
from jax import lax
from jax.experimental import pallas as pl
from jax.experimental.pallas import tpu as pltpu
from jax.experimental.pallas import tpu_sc as plsc

N_NODES = 10000
DEG = 16
D = 256
B = 10000
BP = 10240
NC = 2
NS = 16
NW = NC * NS
BPW = BP // NW
CH = 16
NCHUNK = BPW // CH


def _sc_gather_kernel(nodes_h, adj_h, emb_h, self_h, nsum_h,
                      idxc, adjc0, adjc1, nidx_all,
                      sbuf0, sbuf1, nbA0, nbA1, nbB0, nbB1, mbuf0, mbuf1,
                      sga0, sga1, sgA0, sgA1, sgB0, sgB1, sgs0, sgs1,
                      swm0, swm1, sws0, sws1):
    cid = lax.axis_index("c")
    sid = lax.axis_index("s")
    wid = sid * NC + cid
    base = wid * BPW

    adjc = (adjc0, adjc1)
    sbuf = (sbuf0, sbuf1)
    nbA = (nbA0, nbA1)
    nbB = (nbB0, nbB1)
    mbuf = (mbuf0, mbuf1)
    sga = (sga0, sga1)
    sgA = (sgA0, sgA1)
    sgB = (sgB0, sgB1)
    sgs = (sgs0, sgs1)
    swm = (swm0, swm1)
    sws = (sws0, sws1)

    pltpu.sync_copy(nodes_h.at[pl.ds(wid * BPW, BPW)], idxc)

    def ids(c):
        return idxc.at[pl.ds(c * CH, CH)]

    def adj_dma(c, k):
        return pltpu.make_async_copy(adj_h.at[ids(c)], adjc[k], sga[k])

    def stage_nidx(c, k):
        buf = adjc[k]
        for r in range(CH):
            nidx_all[2 * c + r // 8, pl.ds((r % 8) * DEG, DEG)] = \
                buf[r, pl.ds(0, DEG)]

    def gA_dma(c, k):
        return pltpu.make_async_copy(emb_h.at[nidx_all.at[2 * c]], nbA[k],
                                     sgA[k])

    def gB_dma(c, k):
        return pltpu.make_async_copy(emb_h.at[nidx_all.at[2 * c + 1]], nbB[k],
                                     sgB[k])

    def gs_dma(c, k):
        return pltpu.make_async_copy(emb_h.at[ids(c)], sbuf[k], sgs[k])

    def wm_dma(c, k):
        return pltpu.make_async_copy(
            mbuf[k], nsum_h.at[pl.ds(base + c * CH, CH)], swm[k])

    def ws_dma(c, k):
        return pltpu.make_async_copy(
            sbuf[k], self_h.at[pl.ds(base + c * CH, CH)], sws[k])

    def reduce_chunk(k):
        a_buf, b_buf, mb = nbA[k], nbB[k], mbuf[k]
        bc = lambda v: lax.bitcast_convert_type(v, jnp.float32)

        def row(r, carry):
            for nb, ro in ((a_buf, 0), (b_buf, 8)):
                for kk in range(8):
                    cs = pl.ds(kk * 16, 16)
                    w = nb[r * DEG, cs]
                    alo = bc(w << 16)
                    ahi = bc(w)
                    for j in range(1, DEG):
                        w = nb[r * DEG + j, cs]
                        alo = alo + bc(w << 16)
                        ahi = ahi + bc(w)
                    mb[r + ro, cs] = alo
                    mb[r + ro, pl.ds(128 + kk * 16, 16)] = ahi
            return carry
        lax.fori_loop(0, 8, row, 0)

    adj_dma(0, 0).start()
    adj_dma(0, 0).wait()
    stage_nidx(0, 0)
    adj_dma(1, 1).start()
    gA_dma(0, 0).start()
    gB_dma(0, 0).start()
    gs_dma(0, 0).start()

    def body(c, k):
        kn = 1 - k
        @pl.when(c + 1 < NCHUNK)
        def _():
            adj_dma(c + 1, kn).wait()
            stage_nidx(c + 1, kn)

            @pl.when(c >= 1)
            def _():
                ws_dma(c - 1, kn).wait()
            gA_dma(c + 1, kn).start()
            gB_dma(c + 1, kn).start()
            gs_dma(c + 1, kn).start()

        @pl.when(c + 2 < NCHUNK)
        def _():
            adj_dma(c + 2, k).start()

        gA_dma(c, k).wait()
        gB_dma(c, k).wait()
        gs_dma(c, k).wait()
        ws_dma(c, k).start()

        @pl.when(c >= 2)
        def _():
            wm_dma(c - 2, k).wait()
        reduce_chunk(k)
        wm_dma(c, k).start()

    def phase2(i, carry):
        c = i * 2
        body(c, 0)
        body(c + 1, 1)
        return carry

    lax.fori_loop(0, NCHUNK // 2, phase2, 0)

    wm_dma(NCHUNK - 2, 0).wait()
    ws_dma(NCHUNK - 2, 0).wait()
    wm_dma(NCHUNK - 1, 1).wait()
    ws_dma(NCHUNK - 1, 1).wait()


def _sc_gather(nodes_p, adj_p, emb_i):
    mesh = plsc.VectorSubcoreMesh(core_axis_name="c", subcore_axis_name="s")
    kern = functools.partial(
        pl.kernel,
        mesh=mesh,
        out_type=(
            jax.ShapeDtypeStruct((BP, 128), jnp.int32),
            jax.ShapeDtypeStruct((BP, D), jnp.float32),
        ),
        scratch_types=[
            pltpu.VMEM((BPW,), jnp.int32),
            pltpu.VMEM((CH, 128), jnp.int32),
            pltpu.VMEM((CH, 128), jnp.int32),
            pltpu.VMEM((2 * NCHUNK, 128), jnp.int32),
            pltpu.VMEM((CH, 128), jnp.int32),
            pltpu.VMEM((CH, 128), jnp.int32),
            pltpu.VMEM((128, 128), jnp.int32),
            pltpu.VMEM((128, 128), jnp.int32),
            pltpu.VMEM((128, 128), jnp.int32),
            pltpu.VMEM((128, 128), jnp.int32),
            pltpu.VMEM((CH, D), jnp.float32),
            pltpu.VMEM((CH, D), jnp.float32),
        ] + [pltpu.SemaphoreType.DMA] * 12,
    )(_sc_gather_kernel)
    return kern(nodes_p, adj_p, emb_i)


def _mm_kernel(x1_ref, x2_ref, w1a_ref, w1b_ref, w2_ref, b_ref, o_ref):
    x = x1_ref[...]
    xlo = lax.bitcast_convert_type(x << 16, jnp.float32).astype(jnp.bfloat16)
    xhi = lax.bitcast_convert_type(x & jnp.int32(-65536), jnp.float32
                                   ).astype(jnp.bfloat16)
    acc = jnp.dot(xlo, w1a_ref[...], preferred_element_type=jnp.float32)
    acc += jnp.dot(xhi, w1b_ref[...], preferred_element_type=jnp.float32)
    acc += jnp.dot(x2_ref[...].astype(jnp.bfloat16), w2_ref[...],
                   preferred_element_type=jnp.float32)
    o_ref[...] = jnp.maximum(acc + b_ref[...], 0.0)


def _tc_matmul(self_i, nsum, w1a, w1b, w2, b2):
    bm = 1000
    grid = (B // bm,)
    return pl.pallas_call(
        _mm_kernel,
        grid=grid,
        in_specs=[
            pl.BlockSpec((bm, D // 2), lambda i: (i, 0)),
            pl.BlockSpec((bm, D), lambda i: (i, 0)),
            pl.BlockSpec((D // 2, D), lambda i: (0, 0)),
            pl.BlockSpec((D // 2, D), lambda i: (0, 0)),
            pl.BlockSpec((D, D), lambda i: (0, 0)),
            pl.BlockSpec((1, D), lambda i: (0, 0)),
        ],
        out_specs=pl.BlockSpec((bm, D), lambda i: (i, 0)),
        out_shape=jax.ShapeDtypeStruct((B, D), jnp.float32),
    )(self_i, nsum, w1a, w1b, w2, b2)


def kernel(nodes, adj, emb, W, b):
    nodes_p = jnp.pad(nodes.astype(jnp.int32), (0, BP - B))
    adj_p = jnp.pad(adj.astype(jnp.int32), ((0, 0), (0, 128 - DEG)))
    u = lax.bitcast_convert_type(emb.astype(jnp.bfloat16), jnp.uint16)
    emb_i = (u[:, :D // 2].astype(jnp.uint32)
             | (u[:, D // 2:].astype(jnp.uint32) << 16)).astype(jnp.int32)
    self_i, nsum = _sc_gather(nodes_p, adj_p, emb_i)
    w1 = W[:D, :].astype(jnp.bfloat16)
    w1a, w1b = w1[:D // 2, :], w1[D // 2:, :]
    w2 = (W[D:, :] * (1.0 / DEG)).astype(jnp.bfloat16)
    return _tc_matmul(self_i, nsum, w1a, w1b, w2, b.reshape(1, D))


# speedup vs baseline: 2.7333x; 1.0549x over previous
import functools

import jax
import jax.numpy as jnp
from jax import lax
from jax.experimental import pallas as pl
from jax.experimental.pallas import tpu as pltpu
from jax.experimental.pallas import tpu_sc as plsc

N_NODES = 10000
DEG = 16
D = 256
B = 10000
BP = 10240
NC = 2
NS = 16
NW = NC * NS
BPW = BP // NW
CH = 16
NCHUNK = BPW // CH


def _sc_gather_kernel(nodes_h, adj_h, emb_h, self_h, nsum_h,
                      idxc, adjc0, adjc1, nidx_all,
                      sbuf0, sbuf1, nbA0, nbA1, nbB0, nbB1, mbuf0, mbuf1,
                      sga0, sga1, sgA0, sgA1, sgB0, sgB1, sgs0, sgs1,
                      swm0, swm1, sws0, sws1):
    cid = lax.axis_index("c")
    sid = lax.axis_index("s")
    wid = sid * NC + cid
    base = wid * BPW

    adjc = (adjc0, adjc1)
    sbuf = (sbuf0, sbuf1)
    nbA = (nbA0, nbA1)
    nbB = (nbB0, nbB1)
    mbuf = (mbuf0, mbuf1)
    sga = (sga0, sga1)
    sgA = (sgA0, sgA1)
    sgB = (sgB0, sgB1)
    sgs = (sgs0, sgs1)
    swm = (swm0, swm1)
    sws = (sws0, sws1)

    pltpu.sync_copy(nodes_h.at[pl.ds(wid * BPW, BPW)], idxc)

    def ids(c):
        return idxc.at[pl.ds(c * CH, CH)]

    def adj_dma(c, k):
        return pltpu.make_async_copy(adj_h.at[ids(c)], adjc[k], sga[k])

    def stage_nidx(c, k):
        buf = adjc[k]
        for r in range(CH):
            nidx_all[2 * c + r // 8, pl.ds((r % 8) * DEG, DEG)] = \
                buf[r, pl.ds(0, DEG)]

    def gA_dma(c, k):
        return pltpu.make_async_copy(emb_h.at[nidx_all.at[2 * c]], nbA[k],
                                     sgA[k])

    def gB_dma(c, k):
        return pltpu.make_async_copy(emb_h.at[nidx_all.at[2 * c + 1]], nbB[k],
                                     sgB[k])

    def gs_dma(c, k):
        return pltpu.make_async_copy(emb_h.at[ids(c)], sbuf[k], sgs[k])

    def wm_dma(c, k):
        return pltpu.make_async_copy(
            mbuf[k], nsum_h.at[pl.ds(base + c * CH, CH)], swm[k])

    def ws_dma(c, k):
        return pltpu.make_async_copy(
            sbuf[k], self_h.at[pl.ds(base + c * CH, CH)], sws[k])

    def reduce_chunk(k):
        a_buf, b_buf, mb = nbA[k], nbB[k], mbuf[k]
        bc = lambda v: lax.bitcast_convert_type(v, jnp.float32)

        def row(r, carry):
            for nb, ro in ((a_buf, 0), (b_buf, 8)):
                ws = [nb[r * DEG, pl.ds(kk * 16, 16)] for kk in range(8)]
                alo = [bc(w << 16) for w in ws]
                ahi = [bc(w) for w in ws]
                for j in range(1, DEG):
                    ws = [nb[r * DEG + j, pl.ds(kk * 16, 16)]
                          for kk in range(8)]
                    alo = [a + bc(w << 16) for a, w in zip(alo, ws)]
                    ahi = [a + bc(w) for a, w in zip(ahi, ws)]
                for kk in range(8):
                    mb[r + ro, pl.ds(kk * 16, 16)] = alo[kk]
                    mb[r + ro, pl.ds(128 + kk * 16, 16)] = ahi[kk]
            return carry
        lax.fori_loop(0, 8, row, 0)

    adj_dma(0, 0).start()
    adj_dma(0, 0).wait()
    stage_nidx(0, 0)
    adj_dma(1, 1).start()
    gA_dma(0, 0).start()
    gB_dma(0, 0).start()
    gs_dma(0, 0).start()

    def body(c, k):
        kn = 1 - k
        @pl.when(c + 1 < NCHUNK)
        def _():
            adj_dma(c + 1, kn).wait()
            stage_nidx(c + 1, kn)

            @pl.when(c >= 1)
            def _():
                ws_dma(c - 1, kn).wait()
            gA_dma(c + 1, kn).start()
            gB_dma(c + 1, kn).start()
            gs_dma(c + 1, kn).start()

        @pl.when(c + 2 < NCHUNK)
        def _():
            adj_dma(c + 2, k).start()

        gA_dma(c, k).wait()
        gB_dma(c, k).wait()
        gs_dma(c, k).wait()
        ws_dma(c, k).start()

        @pl.when(c >= 2)
        def _():
            wm_dma(c - 2, k).wait()
        reduce_chunk(k)
        wm_dma(c, k).start()

    def phase2(i, carry):
        c = i * 2
        body(c, 0)
        body(c + 1, 1)
        return carry

    lax.fori_loop(0, NCHUNK // 2, phase2, 0)

    wm_dma(NCHUNK - 2, 0).wait()
    ws_dma(NCHUNK - 2, 0).wait()
    wm_dma(NCHUNK - 1, 1).wait()
    ws_dma(NCHUNK - 1, 1).wait()


def _sc_gather(nodes_p, adj_p, emb_i):
    mesh = plsc.VectorSubcoreMesh(core_axis_name="c", subcore_axis_name="s")
    kern = functools.partial(
        pl.kernel,
        mesh=mesh,
        out_type=(
            jax.ShapeDtypeStruct((BP, 128), jnp.int32),
            jax.ShapeDtypeStruct((BP, D), jnp.float32),
        ),
        scratch_types=[
            pltpu.VMEM((BPW,), jnp.int32),
            pltpu.VMEM((CH, 128), jnp.int32),
            pltpu.VMEM((CH, 128), jnp.int32),
            pltpu.VMEM((2 * NCHUNK, 128), jnp.int32),
            pltpu.VMEM((CH, 128), jnp.int32),
            pltpu.VMEM((CH, 128), jnp.int32),
            pltpu.VMEM((128, 128), jnp.int32),
            pltpu.VMEM((128, 128), jnp.int32),
            pltpu.VMEM((128, 128), jnp.int32),
            pltpu.VMEM((128, 128), jnp.int32),
            pltpu.VMEM((CH, D), jnp.float32),
            pltpu.VMEM((CH, D), jnp.float32),
        ] + [pltpu.SemaphoreType.DMA] * 12,
    )(_sc_gather_kernel)
    return kern(nodes_p, adj_p, emb_i)


def _mm_kernel(x1_ref, x2_ref, w1a_ref, w1b_ref, w2_ref, b_ref, o_ref):
    x = x1_ref[...]
    xlo = lax.bitcast_convert_type(x << 16, jnp.float32).astype(jnp.bfloat16)
    xhi = lax.bitcast_convert_type(x & jnp.int32(-65536), jnp.float32
                                   ).astype(jnp.bfloat16)
    acc = jnp.dot(xlo, w1a_ref[...], preferred_element_type=jnp.float32)
    acc += jnp.dot(xhi, w1b_ref[...], preferred_element_type=jnp.float32)
    acc += jnp.dot(x2_ref[...].astype(jnp.bfloat16), w2_ref[...],
                   preferred_element_type=jnp.float32)
    o_ref[...] = jnp.maximum(acc + b_ref[...], 0.0)


def _tc_matmul(self_i, nsum, w1a, w1b, w2, b2):
    bm = 1000
    grid = (B // bm,)
    return pl.pallas_call(
        _mm_kernel,
        grid=grid,
        in_specs=[
            pl.BlockSpec((bm, D // 2), lambda i: (i, 0)),
            pl.BlockSpec((bm, D), lambda i: (i, 0)),
            pl.BlockSpec((D // 2, D), lambda i: (0, 0)),
            pl.BlockSpec((D // 2, D), lambda i: (0, 0)),
            pl.BlockSpec((D, D), lambda i: (0, 0)),
            pl.BlockSpec((1, D), lambda i: (0, 0)),
        ],
        out_specs=pl.BlockSpec((bm, D), lambda i: (i, 0)),
        out_shape=jax.ShapeDtypeStruct((B, D), jnp.float32),
    )(self_i, nsum, w1a, w1b, w2, b2)


def kernel(nodes, adj, emb, W, b):
    nodes_p = jnp.pad(nodes.astype(jnp.int32), (0, BP - B))
    adj_p = jnp.pad(adj.astype(jnp.int32), ((0, 0), (0, 128 - DEG)))
    u = lax.bitcast_convert_type(emb.astype(jnp.bfloat16), jnp.uint16)
    emb_i = (u[:, :D // 2].astype(jnp.uint32)
             | (u[:, D // 2:].astype(jnp.uint32) << 16)).astype(jnp.int32)
    self_i, nsum = _sc_gather(nodes_p, adj_p, emb_i)
    w1 = W[:D, :].astype(jnp.bfloat16)
    w1a, w1b = w1[:D // 2, :], w1[D // 2:, :]
    w2 = (W[D:, :] * (1.0 / DEG)).astype(jnp.bfloat16)
    return _tc_matmul(self_i, nsum, w1a, w1b, w2, b.reshape(1, D))


# speedup vs baseline: 2.7814x; 1.0176x over previous
import functools

import jax
import jax.numpy as jnp
from jax import lax
from jax.experimental import pallas as pl
from jax.experimental.pallas import tpu as pltpu
from jax.experimental.pallas import tpu_sc as plsc

N_NODES = 10000
DEG = 16
D = 256
B = 10000
BP = 10240
NC = 2
NS = 16
NW = NC * NS
BPW = BP // NW
CH = 16
NCHUNK = BPW // CH


def _sc_gather_kernel(nodes_h, adj_h, emb_h, self_h, nsum_h,
                      idxc, adjc0, adjc1, nidx_all,
                      sbuf0, sbuf1, nbA0, nbA1, nbB0, nbB1, mbuf0, mbuf1,
                      sga0, sga1, sgA0, sgA1, sgB0, sgB1, sgs0, sgs1,
                      swm0, swm1, sws0, sws1):
    cid = lax.axis_index("c")
    sid = lax.axis_index("s")
    wid = sid * NC + cid
    base = wid * BPW

    adjc = (adjc0, adjc1)
    sbuf = (sbuf0, sbuf1)
    nbA = (nbA0, nbA1)
    nbB = (nbB0, nbB1)
    mbuf = (mbuf0, mbuf1)
    sga = (sga0, sga1)
    sgA = (sgA0, sgA1)
    sgB = (sgB0, sgB1)
    sgs = (sgs0, sgs1)
    swm = (swm0, swm1)
    sws = (sws0, sws1)

    pltpu.sync_copy(nodes_h.at[pl.ds(wid * BPW, BPW)], idxc)

    def ids(c):
        return idxc.at[pl.ds(c * CH, CH)]

    def adj_dma(c, k):
        return pltpu.make_async_copy(adj_h.at[ids(c)], adjc[k], sga[k])

    def stage_nidx(c, k):
        buf = adjc[k]
        for r in range(CH):
            nidx_all[2 * c + r // 8, pl.ds((r % 8) * DEG, DEG)] = \
                buf[r, pl.ds(0, DEG)]

    def gA_dma(c, k):
        return pltpu.make_async_copy(emb_h.at[nidx_all.at[2 * c]], nbA[k],
                                     sgA[k])

    def gB_dma(c, k):
        return pltpu.make_async_copy(emb_h.at[nidx_all.at[2 * c + 1]], nbB[k],
                                     sgB[k])

    def gs_dma(c, k):
        return pltpu.make_async_copy(emb_h.at[ids(c)], sbuf[k], sgs[k])

    def wm_dma(c, k):
        return pltpu.make_async_copy(
            mbuf[k], nsum_h.at[pl.ds(base + c * CH, CH)], swm[k])

    def ws_dma(c, k):
        return pltpu.make_async_copy(
            sbuf[k], self_h.at[pl.ds(base + c * CH, CH)], sws[k])

    def reduce_chunk(k):
        a_buf, b_buf, mb = nbA[k], nbB[k], mbuf[k]
        bc = lambda v: lax.bitcast_convert_type(v, jnp.float32)

        def row(r, carry):
            for nb, ro in ((a_buf, 0), (b_buf, 8)):
                ws = [nb[r * DEG, pl.ds(kk * 16, 16)] for kk in range(8)]
                alo = [bc(w << 16) for w in ws]
                ahi = [bc(w) for w in ws]
                for j in range(1, DEG):
                    ws = [nb[r * DEG + j, pl.ds(kk * 16, 16)]
                          for kk in range(8)]
                    alo = [a + bc(w << 16) for a, w in zip(alo, ws)]
                    ahi = [a + bc(w) for a, w in zip(ahi, ws)]
                for kk in range(8):
                    mb[r + ro, pl.ds(kk * 16, 16)] = alo[kk]
                    mb[r + ro, pl.ds(128 + kk * 16, 16)] = ahi[kk]
            return carry
        lax.fori_loop(0, 8, row, 0)

    adj_dma(0, 0).start()
    adj_dma(0, 0).wait()
    stage_nidx(0, 0)
    adj_dma(1, 1).start()
    gA_dma(0, 0).start()
    gB_dma(0, 0).start()
    gs_dma(0, 0).start()

    def body(c, k):
        kn = 1 - k
        @pl.when(c + 1 < NCHUNK)
        def _():
            adj_dma(c + 1, kn).wait()
            stage_nidx(c + 1, kn)

            @pl.when(c >= 1)
            def _():
                ws_dma(c - 1, kn).wait()
            gA_dma(c + 1, kn).start()
            gB_dma(c + 1, kn).start()
            gs_dma(c + 1, kn).start()

        @pl.when(c + 2 < NCHUNK)
        def _():
            adj_dma(c + 2, k).start()

        gA_dma(c, k).wait()
        gB_dma(c, k).wait()
        gs_dma(c, k).wait()
        ws_dma(c, k).start()

        @pl.when(c >= 2)
        def _():
            wm_dma(c - 2, k).wait()
        reduce_chunk(k)
        wm_dma(c, k).start()

    def phase2(i, carry):
        c = i * 2
        body(c, 0)
        body(c + 1, 1)
        return carry

    lax.fori_loop(0, NCHUNK // 2, phase2, 0)

    wm_dma(NCHUNK - 2, 0).wait()
    ws_dma(NCHUNK - 2, 0).wait()
    wm_dma(NCHUNK - 1, 1).wait()
    ws_dma(NCHUNK - 1, 1).wait()


def _sc_gather(nodes_p, adj_p, emb_i):
    mesh = plsc.VectorSubcoreMesh(core_axis_name="c", subcore_axis_name="s")
    kern = functools.partial(
        pl.kernel,
        mesh=mesh,
        out_type=(
            jax.ShapeDtypeStruct((BP, 128), jnp.int32),
            jax.ShapeDtypeStruct((BP, D), jnp.float32),
        ),
        scratch_types=[
            pltpu.VMEM((BPW,), jnp.int32),
            pltpu.VMEM((CH, 128), jnp.int32),
            pltpu.VMEM((CH, 128), jnp.int32),
            pltpu.VMEM((2 * NCHUNK, 128), jnp.int32),
            pltpu.VMEM((CH, 128), jnp.int32),
            pltpu.VMEM((CH, 128), jnp.int32),
            pltpu.VMEM((128, 128), jnp.int32),
            pltpu.VMEM((128, 128), jnp.int32),
            pltpu.VMEM((128, 128), jnp.int32),
            pltpu.VMEM((128, 128), jnp.int32),
            pltpu.VMEM((CH, D), jnp.float32),
            pltpu.VMEM((CH, D), jnp.float32),
        ] + [pltpu.SemaphoreType.DMA] * 12,
    )(_sc_gather_kernel)
    return kern(nodes_p, adj_p, emb_i)


def _mm_kernel(x1_ref, x2_ref, w1a_ref, w1b_ref, w2_ref, b_ref, o_ref):
    x = x1_ref[...]
    xlo = lax.bitcast_convert_type(x << 16, jnp.float32).astype(jnp.bfloat16)
    xhi = lax.bitcast_convert_type(x & jnp.int32(-65536), jnp.float32
                                   ).astype(jnp.bfloat16)
    acc = jnp.dot(xlo, w1a_ref[...], preferred_element_type=jnp.float32)
    acc += jnp.dot(xhi, w1b_ref[...], preferred_element_type=jnp.float32)
    acc += jnp.dot(x2_ref[...].astype(jnp.bfloat16), w2_ref[...],
                   preferred_element_type=jnp.float32)
    o_ref[...] = jnp.maximum(acc + b_ref[...], 0.0)


def _tc_matmul(self_i, nsum, w1a, w1b, w2, b2):
    bm = 2000
    grid = (B // bm,)
    return pl.pallas_call(
        _mm_kernel,
        grid=grid,
        in_specs=[
            pl.BlockSpec((bm, D // 2), lambda i: (i, 0)),
            pl.BlockSpec((bm, D), lambda i: (i, 0)),
            pl.BlockSpec((D // 2, D), lambda i: (0, 0)),
            pl.BlockSpec((D // 2, D), lambda i: (0, 0)),
            pl.BlockSpec((D, D), lambda i: (0, 0)),
            pl.BlockSpec((1, D), lambda i: (0, 0)),
        ],
        out_specs=pl.BlockSpec((bm, D), lambda i: (i, 0)),
        out_shape=jax.ShapeDtypeStruct((B, D), jnp.float32),
    )(self_i, nsum, w1a, w1b, w2, b2)


def kernel(nodes, adj, emb, W, b):
    nodes_p = jnp.pad(nodes.astype(jnp.int32), (0, BP - B))
    adj_p = jnp.pad(adj.astype(jnp.int32), ((0, 0), (0, 128 - DEG)))
    u = lax.bitcast_convert_type(emb.astype(jnp.bfloat16), jnp.uint16)
    emb_i = (u[:, :D // 2].astype(jnp.uint32)
             | (u[:, D // 2:].astype(jnp.uint32) << 16)).astype(jnp.int32)
    self_i, nsum = _sc_gather(nodes_p, adj_p, emb_i)
    w1 = W[:D, :].astype(jnp.bfloat16)
    w1a, w1b = w1[:D // 2, :], w1[D // 2:, :]
    w2 = (W[D:, :] * (1.0 / DEG)).astype(jnp.bfloat16)
    return _tc_matmul(self_i, nsum, w1a, w1b, w2, b.reshape(1, D))


# speedup vs baseline: 2.8659x; 1.0304x over previous
import functools

import jax
import jax.numpy as jnp
from jax import lax
from jax.experimental import pallas as pl
from jax.experimental.pallas import tpu as pltpu
from jax.experimental.pallas import tpu_sc as plsc

N_NODES = 10000
DEG = 16
D = 256
B = 10000
BP = 10240
NC = 2
NS = 16
NW = NC * NS
BPW = BP // NW
CH = 16
NCHUNK = BPW // CH


def _sc_gather_kernel(nodes_h, adj_h, emb_h, self_h, nsum_h,
                      idxc, adjc0, adjc1, nidx_all,
                      sbuf0, sbuf1, nbA0, nbA1, nbB0, nbB1, mbuf0, mbuf1,
                      sga0, sga1, sgA0, sgA1, sgB0, sgB1, sgs0, sgs1,
                      swm0, swm1, sws0, sws1):
    cid = lax.axis_index("c")
    sid = lax.axis_index("s")
    wid = sid * NC + cid
    base = wid * BPW

    adjc = (adjc0, adjc1)
    sbuf = (sbuf0, sbuf1)
    nbA = (nbA0, nbA1)
    nbB = (nbB0, nbB1)
    mbuf = (mbuf0, mbuf1)
    sga = (sga0, sga1)
    sgA = (sgA0, sgA1)
    sgB = (sgB0, sgB1)
    sgs = (sgs0, sgs1)
    swm = (swm0, swm1)
    sws = (sws0, sws1)

    pltpu.sync_copy(nodes_h.at[pl.ds(wid * BPW, BPW)], idxc)

    def ids(c):
        return idxc.at[pl.ds(c * CH, CH)]

    def adj_dma(c, k):
        return pltpu.make_async_copy(adj_h.at[ids(c)], adjc[k], sga[k])

    def stage_nidx(c, k):
        buf = adjc[k]
        for r in range(CH):
            nidx_all[2 * c + r // 8, pl.ds((r % 8) * DEG, DEG)] = \
                buf[r, pl.ds(0, DEG)]

    def gA_dma(c, k):
        return pltpu.make_async_copy(emb_h.at[nidx_all.at[2 * c]], nbA[k],
                                     sgA[k])

    def gB_dma(c, k):
        return pltpu.make_async_copy(emb_h.at[nidx_all.at[2 * c + 1]], nbB[k],
                                     sgB[k])

    def gs_dma(c, k):
        return pltpu.make_async_copy(emb_h.at[ids(c)], sbuf[k], sgs[k])

    def wm_dma(c, k):
        return pltpu.make_async_copy(
            mbuf[k], nsum_h.at[pl.ds(base + c * CH, CH)], swm[k])

    def ws_dma(c, k):
        return pltpu.make_async_copy(
            sbuf[k], self_h.at[pl.ds(base + c * CH, CH)], sws[k])

    def reduce_chunk(k):
        a_buf, b_buf, mb = nbA[k], nbB[k], mbuf[k]
        bc = lambda v: lax.bitcast_convert_type(v, jnp.float32)

        def row(r, carry):
            for nb, ro in ((a_buf, 0), (b_buf, 8)):
                ws = [nb[r * DEG, pl.ds(kk * 16, 16)] for kk in range(8)]
                alo = [bc(w << 16) for w in ws]
                ahi = [bc(w) for w in ws]
                for j in range(1, DEG):
                    ws = [nb[r * DEG + j, pl.ds(kk * 16, 16)]
                          for kk in range(8)]
                    alo = [a + bc(w << 16) for a, w in zip(alo, ws)]
                    ahi = [a + bc(w) for a, w in zip(ahi, ws)]
                for kk in range(8):
                    mb[r + ro, pl.ds(kk * 16, 16)] = alo[kk]
                    mb[r + ro, pl.ds(128 + kk * 16, 16)] = ahi[kk]
            return carry
        lax.fori_loop(0, 8, row, 0)

    adj_dma(0, 0).start()
    adj_dma(0, 0).wait()
    stage_nidx(0, 0)
    adj_dma(1, 1).start()
    gA_dma(0, 0).start()
    gB_dma(0, 0).start()
    gs_dma(0, 0).start()

    def body(c, k):
        kn = 1 - k
        @pl.when(c + 1 < NCHUNK)
        def _():
            adj_dma(c + 1, kn).wait()
            stage_nidx(c + 1, kn)

            @pl.when(c >= 1)
            def _():
                ws_dma(c - 1, kn).wait()
            gA_dma(c + 1, kn).start()
            gB_dma(c + 1, kn).start()
            gs_dma(c + 1, kn).start()

        @pl.when(c + 2 < NCHUNK)
        def _():
            adj_dma(c + 2, k).start()

        gA_dma(c, k).wait()
        gB_dma(c, k).wait()
        gs_dma(c, k).wait()
        ws_dma(c, k).start()

        @pl.when(c >= 2)
        def _():
            wm_dma(c - 2, k).wait()
        reduce_chunk(k)
        wm_dma(c, k).start()

    def phase2(i, carry):
        c = i * 2
        body(c, 0)
        body(c + 1, 1)
        return carry

    lax.fori_loop(0, NCHUNK // 2, phase2, 0)

    wm_dma(NCHUNK - 2, 0).wait()
    ws_dma(NCHUNK - 2, 0).wait()
    wm_dma(NCHUNK - 1, 1).wait()
    ws_dma(NCHUNK - 1, 1).wait()


def _sc_gather(nodes_p, adj_p, emb_i):
    mesh = plsc.VectorSubcoreMesh(core_axis_name="c", subcore_axis_name="s")
    kern = functools.partial(
        pl.kernel,
        mesh=mesh,
        out_type=(
            jax.ShapeDtypeStruct((BP, 128), jnp.int32),
            jax.ShapeDtypeStruct((BP, D), jnp.float32),
        ),
        scratch_types=[
            pltpu.VMEM((BPW,), jnp.int32),
            pltpu.VMEM((CH, 128), jnp.int32),
            pltpu.VMEM((CH, 128), jnp.int32),
            pltpu.VMEM((2 * NCHUNK, 128), jnp.int32),
            pltpu.VMEM((CH, 128), jnp.int32),
            pltpu.VMEM((CH, 128), jnp.int32),
            pltpu.VMEM((128, 128), jnp.int32),
            pltpu.VMEM((128, 128), jnp.int32),
            pltpu.VMEM((128, 128), jnp.int32),
            pltpu.VMEM((128, 128), jnp.int32),
            pltpu.VMEM((CH, D), jnp.float32),
            pltpu.VMEM((CH, D), jnp.float32),
        ] + [pltpu.SemaphoreType.DMA] * 12,
    )(_sc_gather_kernel)
    return kern(nodes_p, adj_p, emb_i)


def _mm_kernel(x1_ref, x2_ref, w_ref, b_ref, o_ref):
    x = x1_ref[...]
    xlo = lax.bitcast_convert_type(x << 16, jnp.float32).astype(jnp.bfloat16)
    xhi = lax.bitcast_convert_type(x & jnp.int32(-65536), jnp.float32
                                   ).astype(jnp.bfloat16)
    w = w_ref[...]
    acc = jnp.dot(xlo, w[:D // 2].astype(jnp.bfloat16),
                  preferred_element_type=jnp.float32)
    acc += jnp.dot(xhi, w[D // 2:D].astype(jnp.bfloat16),
                   preferred_element_type=jnp.float32)
    acc += jnp.dot(x2_ref[...].astype(jnp.bfloat16),
                   (w[D:] * (1.0 / DEG)).astype(jnp.bfloat16),
                   preferred_element_type=jnp.float32)
    o_ref[...] = jnp.maximum(acc + b_ref[...], 0.0)


def _tc_matmul(self_i, nsum, w, b2):
    bm = 2000
    grid = (B // bm,)
    return pl.pallas_call(
        _mm_kernel,
        grid=grid,
        in_specs=[
            pl.BlockSpec((bm, D // 2), lambda i: (i, 0)),
            pl.BlockSpec((bm, D), lambda i: (i, 0)),
            pl.BlockSpec((2 * D, D), lambda i: (0, 0)),
            pl.BlockSpec((1, D), lambda i: (0, 0)),
        ],
        out_specs=pl.BlockSpec((bm, D), lambda i: (i, 0)),
        out_shape=jax.ShapeDtypeStruct((B, D), jnp.float32),
    )(self_i, nsum, w, b2)


def kernel(nodes, adj, emb, W, b):
    nodes_p = jnp.pad(nodes.astype(jnp.int32), (0, BP - B))
    adj_p = jnp.pad(adj.astype(jnp.int32), ((0, 0), (0, 128 - DEG)))
    u = lax.bitcast_convert_type(emb.astype(jnp.bfloat16), jnp.uint16)
    emb_i = (u[:, :D // 2].astype(jnp.uint32)
             | (u[:, D // 2:].astype(jnp.uint32) << 16)).astype(jnp.int32)
    self_i, nsum = _sc_gather(nodes_p, adj_p, emb_i)
    return _tc_matmul(self_i, nsum, W, b.reshape(1, D))
